# baseline (speedup 1.0000x reference)
import functools

import jax
import jax.numpy as jnp
from jax import lax
from jax.experimental import pallas as pl
from jax.experimental.pallas import tpu as pltpu
from jax.experimental.pallas import tpu_sc as plsc

_B = 2
_NW = 8
_NH = 16
_L = 256
_C = 512
_NSEL = 2 * _NH
_LANES = 16
_HALF = 128


def _xlane_max(scratch, v, lane):
    for m in (8, 4, 2, 1):
        scratch[...] = v
        v = jnp.maximum(v, plsc.load_gather(scratch, [lane ^ m]))
    return v


def _xlane_min_i32(scratch, v, lane):
    for m in (8, 4, 2, 1):
        scratch[...] = v
        v = jnp.minimum(v, plsc.load_gather(scratch, [lane ^ m]))
    return v


def _scfull_body(a0_hbm, a1_hbm, xflat_hbm, out_hbm,
                 bufs, idxbuf, gbuf, sumbuf, fvec, ivec, shared,
                 sem0, sem1, gsem):
    c = lax.axis_index("c")
    s = lax.axis_index("s")
    sems = (sem0, sem1)
    nchunks = 2 * _NW * 2

    def start(k):
        a = k // (_NW * 2)
        w = (k // 2) % _NW
        half = k % 2
        src_ref = a0_hbm if a == 0 else a1_hbm
        m = (c * _NW + w) * _NH + s
        src = src_ref.at[pl.ds(m, 1), pl.ds(half * _HALF, _HALF)]
        return pltpu.async_copy(src, bufs.at[pl.ds(k % 2, 1)], sems[k % 2])

    lane = lax.broadcasted_iota(jnp.int32, (_LANES,), 0)
    nch = _L // _LANES
    row_ids = []
    cp = start(0)
    for a in range(2):
        acc = tuple(jnp.zeros((_LANES,), jnp.float32) for _ in range(nch))
        for kk in range(2 * _NW):
            k = a * 2 * _NW + kk
            nxt = start(k + 1) if k + 1 < nchunks else None
            cp.wait()
            kb = k % 2

            def body(r, carry):
                return tuple(
                    carry[ch] + bufs[kb, r, pl.ds(ch * _LANES, _LANES)]
                    for ch in range(nch))

            acc = lax.fori_loop(0, _HALF, body, acc)
            cp = nxt

        m16 = acc[0]
        for ch in range(1, nch):
            m16 = jnp.maximum(m16, acc[ch])
        gm = _xlane_max(fvec, m16, lane)
        cmin = jnp.full((_LANES,), _L, jnp.int32)
        for ch in range(nch):
            cmin = jnp.minimum(
                cmin, jnp.where(acc[ch] == gm, lane + ch * _LANES, _L))
        best = _xlane_min_i32(ivec, cmin, lane)
        row_ids.append(best + c * _L)

    vec = jnp.where(lane == 0, row_ids[0],
                    jnp.where(lane == 1, row_ids[1], 0))
    idxbuf[...] = vec
    pltpu.async_copy(xflat_hbm.at[idxbuf], gbuf, gsem).wait()

    for ch in range(_C // _LANES):
        d = pl.ds(ch * _LANES, _LANES)
        sumbuf[0, d] = (gbuf[0, d] + gbuf[1, d]) * (1.0 / _NSEL)
    pltpu.sync_copy(sumbuf, shared.at[pl.ds(s, 1)])
    plsc.subcore_barrier()

    @pl.when(s == 0)
    def _reduce():
        pltpu.sync_copy(shared, gbuf)
        for ch in range(_C // _LANES):
            d = pl.ds(ch * _LANES, _LANES)
            acc16 = gbuf[0, d]
            for row in range(1, _NH):
                acc16 = acc16 + gbuf[row, d]
            sumbuf[0, d] = acc16
        pltpu.sync_copy(sumbuf, out_hbm.at[pl.ds(c, 1)])


def kernel(x, attn0, attn1):
    a0f = attn0.reshape(_B * _NW * _NH, _L, _L)
    a1f = attn1.reshape(_B * _NW * _NH, _L, _L)
    xflat = x.reshape(_B * _L, _C)
    mesh = plsc.VectorSubcoreMesh(core_axis_name="c", subcore_axis_name="s")
    f = pl.kernel(
        _scfull_body,
        mesh=mesh,
        out_type=jax.ShapeDtypeStruct((_B, _C), jnp.float32),
        scratch_types=[
            pltpu.VMEM((2, _HALF, _L), jnp.float32),
            pltpu.VMEM((_LANES,), jnp.int32),
            pltpu.VMEM((_NH, _C), jnp.float32),
            pltpu.VMEM((1, _C), jnp.float32),
            pltpu.VMEM((_LANES,), jnp.float32),
            pltpu.VMEM((_LANES,), jnp.int32),
            pltpu.VMEM_SHARED((_NH, _C), jnp.float32),
            pltpu.SemaphoreType.DMA,
            pltpu.SemaphoreType.DMA,
            pltpu.SemaphoreType.DMA,
        ],
        compiler_params=pltpu.CompilerParams(needs_layout_passes=False),
    )
    return f(a0f, a1f, xflat)


# speedup vs baseline: 2.2232x; 2.2232x over previous
import jax
import jax.numpy as jnp
from jax.experimental import pallas as pl
from jax.experimental.pallas import tpu as pltpu

_B = 2
_NW = 8
_NH = 16
_L = 256
_C = 512
_NSEL = 2 * _NH


def _fsm_body(x_ref, a0_ref, a1_ref, out_ref, acc_ref):
    b = pl.program_id(0)
    w = pl.program_id(1)

    @pl.when((b == 0) & (w == 0))
    def _init():
        acc_ref[...] = jnp.zeros_like(acc_ref)

    for a, ref in enumerate((a0_ref, a1_ref)):
        sums = [jnp.sum(ref[0, h], axis=0, keepdims=True)
                + jnp.sum(ref[1, h], axis=0, keepdims=True)
                for h in range(_NH)]
        colsum = jnp.concatenate(sums, axis=0)
        base = a * (_B * _NH) + b * _NH
        acc_ref[pl.ds(base, _NH), :] += colsum

    @pl.when((b == _B - 1) & (w == _NW // 2 - 1))
    def _finish():
        acc = acc_ref[...]
        maxv = jnp.max(acc, axis=1, keepdims=True)
        iota = jax.lax.broadcasted_iota(jnp.int32, (2 * _B * _NH, _L), 1)
        idx = jnp.min(jnp.where(acc >= maxv, iota, _L), axis=1, keepdims=True)
        onehot = (iota == idx).astype(jnp.float32)
        for bb in range(_B):
            rows = (onehot[bb * _NH:(bb + 1) * _NH]
                    + onehot[_B * _NH + bb * _NH:_B * _NH + (bb + 1) * _NH])
            wgt = jnp.sum(rows, axis=0) * (1.0 / _NSEL)
            xb = x_ref[bb]
            out_ref[bb, :] = jnp.sum(xb * wgt.reshape(_L, 1), axis=0)


def kernel(x, attn0, attn1):
    grid = (_B, _NW // 2)
    return pl.pallas_call(
        _fsm_body,
        grid=grid,
        in_specs=[
            pl.BlockSpec((_B, _L, _C), lambda b, w: (0, 0, 0)),
            pl.BlockSpec((2, _NH, _L, _L),
                         lambda b, w: (b * (_NW // 2) + w, 0, 0, 0)),
            pl.BlockSpec((2, _NH, _L, _L),
                         lambda b, w: (b * (_NW // 2) + w, 0, 0, 0)),
        ],
        out_specs=pl.BlockSpec((_B, _C), lambda b, w: (0, 0)),
        out_shape=jax.ShapeDtypeStruct((_B, _C), jnp.float32),
        scratch_shapes=[pltpu.VMEM((2 * _B * _NH, _L), jnp.float32)],
        compiler_params=pltpu.CompilerParams(
            dimension_semantics=("arbitrary", "arbitrary"),
        ),
    )(x, attn0, attn1)
